# baseline (speedup 1.0000x reference)
import functools

import jax
import jax.numpy as jnp
from jax import lax
from jax.experimental import pallas as pl
from jax.experimental.pallas import tpu as pltpu
from jax.experimental.pallas import tpu_sc as plsc

N_NODES = 10000
N_EDGES = 2000
D_FEAT = 256
N_INC = 160000
Q = 64
NS = 16
KB = 80
NB = 125
PER_TILE = KB * NB
N_ROWS = N_NODES
E_ROWS = N_EDGES
E_SLAB = N_EDGES // NS
N_SLAB = N_NODES // NS
CHUNK = 125

_mesh = plsc.VectorSubcoreMesh(
    core_axis_name="c", subcore_axis_name="s", num_cores=2, num_subcores=NS
)


@functools.partial(
    pl.kernel,
    out_type=jax.ShapeDtypeStruct((N_NODES, D_FEAT), jnp.float32),
    mesh=_mesh,
    compiler_params=pltpu.CompilerParams(use_tc_tiling_on_sc=False),
    scratch_types=[
        pltpu.VMEM((2, KB, Q), jnp.float32),
        pltpu.VMEM((NB, KB), jnp.int32),
        pltpu.VMEM((NB, KB), jnp.int32),
        pltpu.VMEM((2, KB), jnp.int32),
        pltpu.VMEM((KB, 16), jnp.float32),
        pltpu.VMEM((CHUNK, Q), jnp.float32),
        pltpu.VMEM((CHUNK, 16), jnp.float32),
        pltpu.VMEM((CHUNK, Q), jnp.float32),
        pltpu.VMEM((2 * Q,), jnp.float32),
        pltpu.VMEM_SHARED((E_ROWS, Q), jnp.float32),
        pltpu.VMEM_SHARED((N_ROWS, Q), jnp.float32),
        pltpu.VMEM_SHARED((E_ROWS, 16), jnp.float32),
        pltpu.VMEM_SHARED((N_ROWS, 16), jnp.float32),
        pltpu.SemaphoreType.DMA((2,)),
        pltpu.SemaphoreType.DMA((2,)),
        pltpu.SemaphoreType.DMA,
    ],
)
def _hyper_gcn_sc(
    x4, nidx_hbm, eidx_hbm, bias_hbm, out_hbm,
    rows2, nidx_all, eidx_all, nadj, ones_v, wbuf, cbuf, obuf, bias_v,
    e_acc, n_acc, e_cnt, n_cnt, sem_g, sem_s, sem_c,
):
    cid = lax.axis_index("c")
    sid = lax.axis_index("s")
    zeros16 = jnp.zeros((16,), jnp.float32)
    ones16 = jnp.ones((16,), jnp.float32)
    ebase = sid * E_SLAB

    pltpu.sync_copy(nidx_hbm.at[pl.ds(sid * NB, NB)], nidx_all)
    pltpu.sync_copy(eidx_hbm.at[pl.ds(sid * NB, NB)], eidx_all)
    pltpu.sync_copy(bias_hbm.at[pl.ds(cid * 2 * Q, 2 * Q)], bias_v)

    def _fill_ones(r, carry):
        ones_v[r, :] = ones16
        return carry

    lax.fori_loop(0, KB, _fill_ones, 0)

    def _zero_cnt_row(r, carry):
        cbuf[r, :] = zeros16
        return carry

    lax.fori_loop(0, CHUNK, _zero_cnt_row, 0)

    def _stage_loop(stage, rnd, q):
        counts = stage == 1 and rnd == 0

        def adjust(b, g):
            for i in range(KB // 16):
                v = nidx_all[b, pl.ds(i * 16, 16)]
                nadj[g, pl.ds(i * 16, 16)] = (
                    (v >> 3) * 32 + (v & 7) * 2 + (q // 2) * 16 + (q % 2)
                )

        def issue_gather(b, g):
            if stage == 1:
                pltpu.async_copy(x4.at[nadj.at[g]], rows2.at[g], sem_g.at[g])
            else:
                pltpu.async_copy(
                    e_acc.at[eidx_all.at[b]], rows2.at[g], sem_g.at[g]
                )

        def wait_gather(b, g):
            if stage == 1:
                pltpu.make_async_copy(
                    x4.at[nadj.at[g]], rows2.at[g], sem_g.at[g]
                ).wait()
            else:
                pltpu.make_async_copy(
                    e_acc.at[eidx_all.at[b]], rows2.at[g], sem_g.at[g]
                ).wait()

        def issue_scatter(b, g):
            if stage == 1:
                pltpu.async_copy(
                    rows2.at[g], e_acc.at[eidx_all.at[b]], sem_s.at[g], add=True
                )
            else:
                pltpu.async_copy(
                    rows2.at[g], n_acc.at[nidx_all.at[b]], sem_s.at[g], add=True
                )

        def wait_scatter(b, g):
            if stage == 1:
                pltpu.make_async_copy(
                    rows2.at[g], e_acc.at[eidx_all.at[b]], sem_s.at[g]
                ).wait()
            else:
                pltpu.make_async_copy(
                    rows2.at[g], n_acc.at[nidx_all.at[b]], sem_s.at[g]
                ).wait()

        if stage == 1:
            adjust(0, 0)
        issue_gather(0, 0)

        def body(b, carry):
            g = lax.rem(b, 2)
            g1 = lax.rem(b + 1, 2)

            @pl.when(b >= 1)
            def _():
                wait_scatter(b - 1, g1)

            @pl.when(b + 1 < NB)
            def _():
                if stage == 1:
                    adjust(b + 1, g1)
                issue_gather(b + 1, g1)

            if counts:
                pltpu.async_copy(
                    ones_v, n_cnt.at[nidx_all.at[b]], sem_c, add=True
                )
                pltpu.async_copy(
                    ones_v, e_cnt.at[eidx_all.at[b]], sem_c, add=True
                )

            wait_gather(b, g)
            issue_scatter(b, g)
            return carry

        lax.fori_loop(0, NB, body, 0)
        wait_scatter(NB - 1, lax.rem(NB - 1, 2))
        if counts:
            def _drain(i, carry):
                pltpu.make_async_copy(
                    ones_v, n_cnt.at[nidx_all.at[0]], sem_c
                ).wait()
                pltpu.make_async_copy(
                    ones_v, e_cnt.at[eidx_all.at[0]], sem_c
                ).wait()
                return carry

            lax.fori_loop(0, NB, _drain, 0)

    for rnd in range(2):
        q = 2 * cid + rnd

        def _zero_row(r, carry):
            for c in range(Q // 16):
                wbuf[r, pl.ds(c * 16, 16)] = zeros16
            return carry

        lax.fori_loop(0, CHUNK, _zero_row, 0)
        pltpu.sync_copy(wbuf, e_acc.at[pl.ds(ebase, CHUNK)])
        if rnd == 0:
            pltpu.sync_copy(cbuf, e_cnt.at[pl.ds(ebase, CHUNK)])

        def _zero_nodes(j, carry):
            nb = sid * N_SLAB + j * CHUNK
            pltpu.sync_copy(wbuf, n_acc.at[pl.ds(nb, CHUNK)])
            if rnd == 0:
                pltpu.sync_copy(cbuf, n_cnt.at[pl.ds(nb, CHUNK)])
            return carry

        lax.fori_loop(0, N_SLAB // CHUNK, _zero_nodes, 0)
        plsc.subcore_barrier()

        _stage_loop(1, rnd, q)
        plsc.subcore_barrier()

        pltpu.sync_copy(e_acc.at[pl.ds(ebase, E_SLAB)], wbuf.at[pl.ds(0, E_SLAB)])
        pltpu.sync_copy(e_cnt.at[pl.ds(ebase, E_SLAB)], cbuf.at[pl.ds(0, E_SLAB)])

        def _scale_edge(r, carry):
            cnt = cbuf[r, :]
            rs = jnp.where(cnt > 0.0, 1.0 / cnt, 0.0)
            for c in range(Q // 16):
                wbuf[r, pl.ds(c * 16, 16)] = wbuf[r, pl.ds(c * 16, 16)] * rs
            return carry

        lax.fori_loop(0, E_SLAB, _scale_edge, 0)
        pltpu.sync_copy(wbuf.at[pl.ds(0, E_SLAB)], e_acc.at[pl.ds(ebase, E_SLAB)])
        plsc.subcore_barrier()

        _stage_loop(2, rnd, q)
        plsc.subcore_barrier()

        def _final(j, carry):
            nb = sid * N_SLAB + j * CHUNK
            pltpu.sync_copy(n_acc.at[pl.ds(nb, CHUNK)], wbuf)
            pltpu.sync_copy(n_cnt.at[pl.ds(nb, CHUNK)], cbuf)

            def _row(r, c2):
                cnt = cbuf[r, :]
                rs = jnp.where(cnt > 0.0, 1.0 / cnt, 0.0)
                for c in range(Q // 16):
                    v = (
                        wbuf[r, pl.ds(c * 16, 16)] * rs
                        + bias_v[pl.ds(rnd * Q + c * 16, 16)]
                    )
                    obuf[r, pl.ds(c * 16, 16)] = jnp.maximum(v, 0.01 * v)
                return c2

            lax.fori_loop(0, CHUNK, _row, 0)
            pltpu.sync_copy(
                obuf, out_hbm.at[pl.ds(nb, CHUNK), pl.ds(q * Q, Q)]
            )
            return carry

        lax.fori_loop(0, N_SLAB // CHUNK, _final, 0)
        if rnd == 0:
            plsc.subcore_barrier()


@jax.jit
def kernel(x, hyperedge_index, bias):
    x4 = (
        x.reshape(N_NODES // 8, 8, 2, 2, Q)
        .transpose(0, 2, 1, 3, 4)
        .reshape(4 * N_NODES, Q)
    )
    nidx = hyperedge_index[0].reshape(NS * NB, KB)
    eidx = hyperedge_index[1].reshape(NS * NB, KB)
    return _hyper_gcn_sc(x4, nidx, eidx, bias)


# speedup vs baseline: 1.1745x; 1.1745x over previous
import functools

import jax
import jax.numpy as jnp
from jax import lax
from jax.experimental import pallas as pl
from jax.experimental.pallas import tpu as pltpu
from jax.experimental.pallas import tpu_sc as plsc

N_NODES = 10000
N_EDGES = 2000
D_FEAT = 256
N_INC = 160000
Q = 64
NS = 16
KB = 80
NB = 125
PER_TILE = KB * NB
N_ROWS = N_NODES
E_ROWS = N_EDGES
E_SLAB = N_EDGES // NS
N_SLAB = N_NODES // NS
CHUNK = 125

_mesh = plsc.VectorSubcoreMesh(
    core_axis_name="c", subcore_axis_name="s", num_cores=2, num_subcores=NS
)


@functools.partial(
    pl.kernel,
    out_type=jax.ShapeDtypeStruct((N_NODES, D_FEAT), jnp.float32),
    mesh=_mesh,
    compiler_params=pltpu.CompilerParams(use_tc_tiling_on_sc=False),
    scratch_types=[
        pltpu.VMEM((2, KB, Q), jnp.float32),
        pltpu.VMEM((NB, KB), jnp.int32),
        pltpu.VMEM((NB, KB), jnp.int32),
        pltpu.VMEM((2, KB), jnp.int32),
        pltpu.VMEM((KB, 16), jnp.float32),
        pltpu.VMEM((CHUNK, Q), jnp.float32),
        pltpu.VMEM((CHUNK, 16), jnp.float32),
        pltpu.VMEM((CHUNK, Q), jnp.float32),
        pltpu.VMEM((2 * Q,), jnp.float32),
        pltpu.VMEM_SHARED((E_ROWS, Q), jnp.float32),
        pltpu.VMEM_SHARED((N_ROWS, Q), jnp.float32),
        pltpu.VMEM_SHARED((E_ROWS, 16), jnp.float32),
        pltpu.VMEM_SHARED((N_ROWS, 16), jnp.float32),
        pltpu.SemaphoreType.DMA((2,)),
        pltpu.SemaphoreType.DMA((2,)),
        pltpu.SemaphoreType.DMA,
    ],
)
def _hyper_gcn_sc(
    x4, nidx_hbm, eidx_hbm, bias_hbm, out_hbm,
    rows2, nidx_all, eidx_all, nadj, ones_v, wbuf, cbuf, obuf, bias_v,
    e_acc, n_acc, e_cnt, n_cnt, sem_g, sem_s, sem_c,
):
    cid = lax.axis_index("c")
    sid = lax.axis_index("s")
    zeros16 = jnp.zeros((16,), jnp.float32)
    ones16 = jnp.ones((16,), jnp.float32)
    ebase = sid * E_SLAB

    pltpu.sync_copy(nidx_hbm.at[pl.ds(sid * NB, NB)], nidx_all)
    pltpu.sync_copy(eidx_hbm.at[pl.ds(sid * NB, NB)], eidx_all)
    pltpu.sync_copy(bias_hbm.at[pl.ds(cid * 2 * Q, 2 * Q)], bias_v)

    def _fill_ones(r, carry):
        ones_v[r, :] = ones16
        return carry

    lax.fori_loop(0, KB, _fill_ones, 0)

    def _zero_cnt_row(r, carry):
        cbuf[r, :] = zeros16
        return carry

    lax.fori_loop(0, CHUNK, _zero_cnt_row, 0)

    def _stage_loop(stage, rnd, q):
        counts = stage == 1 and rnd == 0

        def adjust(b, g):
            for i in range(KB // 16):
                v = nidx_all[b, pl.ds(i * 16, 16)]
                nadj[g, pl.ds(i * 16, 16)] = v * 4 + q

        def issue_gather(b, g):
            if stage == 1:
                pltpu.async_copy(x4.at[nadj.at[g]], rows2.at[g], sem_g.at[g])
            else:
                pltpu.async_copy(
                    e_acc.at[eidx_all.at[b]], rows2.at[g], sem_g.at[g]
                )

        def wait_gather(b, g):
            if stage == 1:
                pltpu.make_async_copy(
                    x4.at[nadj.at[g]], rows2.at[g], sem_g.at[g]
                ).wait()
            else:
                pltpu.make_async_copy(
                    e_acc.at[eidx_all.at[b]], rows2.at[g], sem_g.at[g]
                ).wait()

        def issue_scatter(b, g):
            if stage == 1:
                pltpu.async_copy(
                    rows2.at[g], e_acc.at[eidx_all.at[b]], sem_s.at[g], add=True
                )
            else:
                pltpu.async_copy(
                    rows2.at[g], n_acc.at[nidx_all.at[b]], sem_s.at[g], add=True
                )

        def wait_scatter(b, g):
            if stage == 1:
                pltpu.make_async_copy(
                    rows2.at[g], e_acc.at[eidx_all.at[b]], sem_s.at[g]
                ).wait()
            else:
                pltpu.make_async_copy(
                    rows2.at[g], n_acc.at[nidx_all.at[b]], sem_s.at[g]
                ).wait()

        if stage == 1:
            adjust(0, 0)
        issue_gather(0, 0)

        def body(b, carry):
            g = lax.rem(b, 2)
            g1 = lax.rem(b + 1, 2)

            @pl.when(b >= 1)
            def _():
                wait_scatter(b - 1, g1)

            @pl.when(b + 1 < NB)
            def _():
                if stage == 1:
                    adjust(b + 1, g1)
                issue_gather(b + 1, g1)

            if counts:
                pltpu.async_copy(
                    ones_v, n_cnt.at[nidx_all.at[b]], sem_c, add=True
                )
                pltpu.async_copy(
                    ones_v, e_cnt.at[eidx_all.at[b]], sem_c, add=True
                )

            wait_gather(b, g)
            issue_scatter(b, g)
            return carry

        lax.fori_loop(0, NB, body, 0)
        wait_scatter(NB - 1, lax.rem(NB - 1, 2))
        if counts:
            def _drain(i, carry):
                pltpu.make_async_copy(
                    ones_v, n_cnt.at[nidx_all.at[0]], sem_c
                ).wait()
                pltpu.make_async_copy(
                    ones_v, e_cnt.at[eidx_all.at[0]], sem_c
                ).wait()
                return carry

            lax.fori_loop(0, NB, _drain, 0)

    for rnd in range(2):
        q = 2 * cid + rnd

        def _zero_row(r, carry):
            for c in range(Q // 16):
                wbuf[r, pl.ds(c * 16, 16)] = zeros16
            return carry

        lax.fori_loop(0, CHUNK, _zero_row, 0)
        pltpu.sync_copy(wbuf, e_acc.at[pl.ds(ebase, CHUNK)])
        if rnd == 0:
            pltpu.sync_copy(cbuf, e_cnt.at[pl.ds(ebase, CHUNK)])

        def _zero_nodes(j, carry):
            nb = sid * N_SLAB + j * CHUNK
            pltpu.sync_copy(wbuf, n_acc.at[pl.ds(nb, CHUNK)])
            if rnd == 0:
                pltpu.sync_copy(cbuf, n_cnt.at[pl.ds(nb, CHUNK)])
            return carry

        lax.fori_loop(0, N_SLAB // CHUNK, _zero_nodes, 0)
        plsc.subcore_barrier()

        _stage_loop(1, rnd, q)
        plsc.subcore_barrier()

        pltpu.sync_copy(e_acc.at[pl.ds(ebase, E_SLAB)], wbuf.at[pl.ds(0, E_SLAB)])
        pltpu.sync_copy(e_cnt.at[pl.ds(ebase, E_SLAB)], cbuf.at[pl.ds(0, E_SLAB)])

        def _scale_edge(r, carry):
            cnt = cbuf[r, :]
            rs = jnp.where(cnt > 0.0, 1.0 / cnt, 0.0)
            for c in range(Q // 16):
                wbuf[r, pl.ds(c * 16, 16)] = wbuf[r, pl.ds(c * 16, 16)] * rs
            return carry

        lax.fori_loop(0, E_SLAB, _scale_edge, 0)
        pltpu.sync_copy(wbuf.at[pl.ds(0, E_SLAB)], e_acc.at[pl.ds(ebase, E_SLAB)])
        plsc.subcore_barrier()

        _stage_loop(2, rnd, q)
        plsc.subcore_barrier()

        def _final(j, carry):
            nb = sid * N_SLAB + j * CHUNK
            pltpu.sync_copy(n_acc.at[pl.ds(nb, CHUNK)], wbuf)
            pltpu.sync_copy(n_cnt.at[pl.ds(nb, CHUNK)], cbuf)

            def _row(r, c2):
                cnt = cbuf[r, :]
                rs = jnp.where(cnt > 0.0, 1.0 / cnt, 0.0)
                for c in range(Q // 16):
                    v = (
                        wbuf[r, pl.ds(c * 16, 16)] * rs
                        + bias_v[pl.ds(rnd * Q + c * 16, 16)]
                    )
                    obuf[r, pl.ds(c * 16, 16)] = jnp.maximum(v, 0.01 * v)
                return c2

            lax.fori_loop(0, CHUNK, _row, 0)
            pltpu.sync_copy(
                obuf, out_hbm.at[pl.ds(nb, CHUNK), pl.ds(q * Q, Q)]
            )
            return carry

        lax.fori_loop(0, N_SLAB // CHUNK, _final, 0)
        if rnd == 0:
            plsc.subcore_barrier()


@jax.jit
def kernel(x, hyperedge_index, bias):
    x4 = x.reshape(4 * N_NODES, Q)
    nidx = hyperedge_index[0].reshape(NS * NB, KB)
    eidx = hyperedge_index[1].reshape(NS * NB, KB)
    return _hyper_gcn_sc(x4, nidx, eidx, bias)
